# baseline (speedup 1.0000x reference)
import functools
import jax
import jax.numpy as jnp
from jax import lax
from jax.experimental import pallas as pl
from jax.experimental.pallas import tpu as pltpu
from jax.experimental.pallas import tpu_sc as plsc


def _sc_gather_coeffs(sa_tab, s1a_tab, t):
  B = t.shape[0]
  T = sa_tab.shape[0]
  info = plsc.get_sparse_core_info()
  nw = info.num_cores * info.num_subcores
  L = info.num_lanes
  b_per_w = B // nw
  mesh = plsc.VectorSubcoreMesh(core_axis_name="c", subcore_axis_name="s")

  @functools.partial(
      pl.kernel,
      out_type=(
          jax.ShapeDtypeStruct((B,), jnp.float32),
          jax.ShapeDtypeStruct((B,), jnp.float32),
      ),
      mesh=mesh,
      scratch_types=[
          pltpu.VMEM((T,), jnp.float32),
          pltpu.VMEM((T,), jnp.float32),
          pltpu.VMEM((b_per_w,), jnp.int32),
          pltpu.VMEM((b_per_w,), jnp.float32),
          pltpu.VMEM((b_per_w,), jnp.float32),
      ],
      compiler_params=pltpu.CompilerParams(needs_layout_passes=False),
  )
  def gather_kernel(sa_hbm, s1a_hbm, t_hbm, osa_hbm, os1a_hbm,
                    sa_v, s1a_v, idx_v, osa_v, os1a_v):
    wid = lax.axis_index("s") * info.num_cores + lax.axis_index("c")
    base = wid * b_per_w
    pltpu.sync_copy(sa_hbm, sa_v)
    pltpu.sync_copy(s1a_hbm, s1a_v)
    pltpu.sync_copy(t_hbm.at[pl.ds(base, b_per_w)], idx_v)
    for j in range(b_per_w // L):
      idx = idx_v[pl.ds(j * L, L)]
      osa_v[pl.ds(j * L, L)] = plsc.load_gather(sa_v, [idx])
      os1a_v[pl.ds(j * L, L)] = plsc.load_gather(s1a_v, [idx])
    pltpu.sync_copy(osa_v, osa_hbm.at[pl.ds(base, b_per_w)])
    pltpu.sync_copy(os1a_v, os1a_hbm.at[pl.ds(base, b_per_w)])

  return gather_kernel(sa_tab, s1a_tab, t)


def _blend_body(sa_ref, s1a_ref, x_ref, n_ref, o_ref):
  o_ref[...] = sa_ref[...] * x_ref[...] + s1a_ref[...] * n_ref[...]


def _tc_blend(sa, s1a, x2d, n2d, blk):
  B, D = x2d.shape
  grid = (B // blk,)
  return pl.pallas_call(
      _blend_body,
      grid=grid,
      in_specs=[
          pl.BlockSpec((blk, 1), lambda i: (i, 0)),
          pl.BlockSpec((blk, 1), lambda i: (i, 0)),
          pl.BlockSpec((blk, D), lambda i: (i, 0)),
          pl.BlockSpec((blk, D), lambda i: (i, 0)),
      ],
      out_specs=pl.BlockSpec((blk, D), lambda i: (i, 0)),
      out_shape=jax.ShapeDtypeStruct((B, D), jnp.float32),
      compiler_params=pltpu.CompilerParams(
          dimension_semantics=("arbitrary",),
      ),
  )(sa, s1a, x2d, n2d)


@jax.jit
def kernel(x_start, noise, t, sqrt_alphas_cumprod, sqrt_one_minus_alphas_cumprod):
  B = x_start.shape[0]
  D = x_start.size // B
  sa_g, s1a_g = _sc_gather_coeffs(
      sqrt_alphas_cumprod, sqrt_one_minus_alphas_cumprod, t)
  x2d = x_start.reshape(B, D)
  n2d = noise.reshape(B, D)
  out = _tc_blend(sa_g.reshape(B, 1), s1a_g.reshape(B, 1), x2d, n2d, blk=64)
  return out.reshape(x_start.shape)


# speedup vs baseline: 1.0103x; 1.0103x over previous
import functools
import jax
import jax.numpy as jnp
from jax import lax
from jax.experimental import pallas as pl
from jax.experimental.pallas import tpu as pltpu
from jax.experimental.pallas import tpu_sc as plsc


def _sc_gather_coeffs(sa_tab, s1a_tab, t):
  B = t.shape[0]
  T = sa_tab.shape[0]
  info = plsc.get_sparse_core_info()
  nw = info.num_cores * info.num_subcores
  L = info.num_lanes
  b_per_w = B // nw
  mesh = plsc.VectorSubcoreMesh(core_axis_name="c", subcore_axis_name="s")

  @functools.partial(
      pl.kernel,
      out_type=(
          jax.ShapeDtypeStruct((B,), jnp.float32),
          jax.ShapeDtypeStruct((B,), jnp.float32),
      ),
      mesh=mesh,
      scratch_types=[
          pltpu.VMEM((T,), jnp.float32),
          pltpu.VMEM((T,), jnp.float32),
          pltpu.VMEM((b_per_w,), jnp.int32),
          pltpu.VMEM((b_per_w,), jnp.float32),
          pltpu.VMEM((b_per_w,), jnp.float32),
      ],
      compiler_params=pltpu.CompilerParams(needs_layout_passes=False),
  )
  def gather_kernel(sa_hbm, s1a_hbm, t_hbm, osa_hbm, os1a_hbm,
                    sa_v, s1a_v, idx_v, osa_v, os1a_v):
    wid = lax.axis_index("s") * info.num_cores + lax.axis_index("c")
    base = wid * b_per_w
    pltpu.sync_copy(sa_hbm, sa_v)
    pltpu.sync_copy(s1a_hbm, s1a_v)
    pltpu.sync_copy(t_hbm.at[pl.ds(base, b_per_w)], idx_v)
    for j in range(b_per_w // L):
      idx = idx_v[pl.ds(j * L, L)]
      osa_v[pl.ds(j * L, L)] = plsc.load_gather(sa_v, [idx])
      os1a_v[pl.ds(j * L, L)] = plsc.load_gather(s1a_v, [idx])
    pltpu.sync_copy(osa_v, osa_hbm.at[pl.ds(base, b_per_w)])
    pltpu.sync_copy(os1a_v, os1a_hbm.at[pl.ds(base, b_per_w)])

  return gather_kernel(sa_tab, s1a_tab, t)


def _blend_body(sa_ref, s1a_ref, x_ref, n_ref, o_ref):
  o_ref[...] = sa_ref[...] * x_ref[...] + s1a_ref[...] * n_ref[...]


def _tc_blend(sa, s1a, x2d, n2d, blk):
  B, D = x2d.shape
  grid = (B // blk,)
  return pl.pallas_call(
      _blend_body,
      grid=grid,
      in_specs=[
          pl.BlockSpec((blk, 1), lambda i: (i, 0)),
          pl.BlockSpec((blk, 1), lambda i: (i, 0)),
          pl.BlockSpec((blk, D), lambda i: (i, 0)),
          pl.BlockSpec((blk, D), lambda i: (i, 0)),
      ],
      out_specs=pl.BlockSpec((blk, D), lambda i: (i, 0)),
      out_shape=jax.ShapeDtypeStruct((B, D), jnp.float32),
      compiler_params=pltpu.CompilerParams(
          dimension_semantics=("arbitrary",),
      ),
  )(sa, s1a, x2d, n2d)


@jax.jit
def kernel(x_start, noise, t, sqrt_alphas_cumprod, sqrt_one_minus_alphas_cumprod):
  B = x_start.shape[0]
  D = x_start.size // B
  sa_g = jnp.take(sqrt_alphas_cumprod, t, axis=0)
  s1a_g = jnp.take(sqrt_one_minus_alphas_cumprod, t, axis=0)
  x2d = x_start.reshape(B, D)
  n2d = noise.reshape(B, D)
  out = _tc_blend(sa_g.reshape(B, 1), s1a_g.reshape(B, 1), x2d, n2d, blk=64)
  return out.reshape(x_start.shape)
